# baseline (speedup 1.0000x reference)
import jax
import jax.numpy as jnp
from jax.experimental import pallas as pl

B = 1024
_HI = jax.lax.Precision.HIGHEST
_CT = (((1,), (1,)), ((), ()))


def _body(xs_ref, w1_ref, b1_ref, w2_ref, b2_ref, w3_ref, b3_ref,
          w4_ref, b4_ref, w5_ref, b5_ref, w6_ref, b6_ref,
          xhat_ref, zs_ref):
    x = xs_ref[:]
    sq = jnp.sum(x * x, axis=1, keepdims=True)
    x_hi = x.astype(jnp.bfloat16).astype(jnp.float32)
    x_lo = x - x_hi
    cat_a = jnp.concatenate([x_hi, x_lo], axis=1)
    cat_b = jnp.concatenate([x_lo, x_hi], axis=1)
    g = (jax.lax.dot_general(cat_a, cat_a, (((1,), (1,)), ((), ())))
         + jax.lax.dot_general(cat_a, cat_b, (((1,), (1,)), ((), ()))))
    d2 = sq + jnp.transpose(sq) - 2.0 * g
    row = jax.lax.broadcasted_iota(jnp.int32, (B, B), 0)
    col = jax.lax.broadcasted_iota(jnp.int32, (B, B), 1)
    d2 = jnp.where(row == col, jnp.float32(1e9), d2)
    rowmin = jnp.min(d2, axis=1, keepdims=True)
    cand = jnp.where(d2 == rowmin, col, B)
    idx = jnp.min(cand, axis=1, keepdims=True)
    onehot = (col == idx).astype(jnp.float32)

    a1_all = jax.lax.dot_general(x, w1_ref[:], _CT)
    a_hi = a1_all.astype(jnp.bfloat16).astype(jnp.float32)
    r1 = a1_all - a_hi
    a_mid = r1.astype(jnp.bfloat16).astype(jnp.float32)
    a_lo = r1 - a_mid
    limbs = jnp.concatenate([a_hi, a_mid, a_lo], axis=1)
    g3 = jax.lax.dot_general(onehot, limbs, (((1,), (0,)), ((), ())))
    a1_nn = (g3[:, 0:64] + g3[:, 64:128]) + g3[:, 128:192]
    a1 = a1_nn + b1_ref[:][None, :]
    t1 = a1_all - a1_nn
    h1 = jnp.maximum(a1, 0.0)
    dt1 = jnp.where(a1 > 0.0, t1, 0.0)

    a2 = jax.lax.dot_general(h1, w2_ref[:], _CT) + b2_ref[:][None, :]
    t2 = jax.lax.dot_general(dt1, w2_ref[:], _CT)
    h2 = jnp.maximum(a2, 0.0)
    dt2 = jnp.where(a2 > 0.0, t2, 0.0)

    z0 = jax.lax.dot_general(h2, w3_ref[:], _CT) + b3_ref[:][None, :]
    gz = jax.lax.dot_general(dt2, w3_ref[:], _CT)
    zs = z0 + gz
    zs_ref[:] = zs

    h4 = jnp.maximum(jax.lax.dot_general(zs, w4_ref[:], _CT) + b4_ref[:][None, :], 0.0)
    h5 = jnp.maximum(jax.lax.dot_general(h4, w5_ref[:], _CT) + b5_ref[:][None, :], 0.0)
    xhat_ref[:] = jax.lax.dot_general(h5, w6_ref[:], _CT) + b6_ref[:][None, :]


def kernel(xs, W1, b1, W2, b2, W3, b3, W4, b4, W5, b5, W6, b6):
    d = xs.shape[1]
    call = pl.pallas_call(
        _body,
        out_shape=(
            jax.ShapeDtypeStruct((B, d), jnp.float32),
            jax.ShapeDtypeStruct((B, W3.shape[0]), jnp.float32),
        ),
    )
    x_hats, zs = call(xs, W1, b1, W2, b2, W3, b3, W4, b4, W5, b5, W6, b6)
    return (x_hats, zs)


# speedup vs baseline: 1.0552x; 1.0552x over previous
import jax
import jax.numpy as jnp
from jax.experimental import pallas as pl

B = 1024
_HI = jax.lax.Precision.HIGHEST
_CT = (((1,), (1,)), ((), ()))


def _body(xs_ref, w1_ref, b1_ref, w2_ref, b2_ref, w3_ref, b3_ref,
          w4_ref, b4_ref, w5_ref, b5_ref, w6_ref, b6_ref,
          xhat_ref, zs_ref):
    x = xs_ref[:]
    sqh = 0.5 * jnp.sum(x * x, axis=1, keepdims=True)
    x_hi = x.astype(jnp.bfloat16).astype(jnp.float32)
    x_lo = x - x_hi
    cat_l = jnp.concatenate([x_hi, x_lo, x_hi], axis=1)
    cat_r = jnp.concatenate([x_hi, x_hi, x_lo], axis=1)
    g = jax.lax.dot_general(cat_l, cat_r, (((1,), (1,)), ((), ())))
    score = jnp.transpose(sqh) - g
    row = jax.lax.broadcasted_iota(jnp.int32, (B, B), 0)
    col = jax.lax.broadcasted_iota(jnp.int32, (B, B), 1)
    score = jnp.where(row == col, jnp.float32(1e9), score)
    rowmin = jnp.min(score, axis=1, keepdims=True)
    cand = jnp.where(score == rowmin, col, B)
    idx = jnp.min(cand, axis=1, keepdims=True)
    onehot = (cand == idx).astype(jnp.float32)

    a1_all = jax.lax.dot_general(x, w1_ref[:], _CT)
    a_hi = a1_all.astype(jnp.bfloat16).astype(jnp.float32)
    r1 = a1_all - a_hi
    a_mid = r1.astype(jnp.bfloat16).astype(jnp.float32)
    a_lo = r1 - a_mid
    limbs = jnp.concatenate([a_hi, a_mid, a_lo], axis=1)
    g3 = jax.lax.dot_general(onehot, limbs, (((1,), (0,)), ((), ())))
    a1_nn = (g3[:, 0:64] + g3[:, 64:128]) + g3[:, 128:192]
    a1 = a1_nn + b1_ref[:][None, :]
    t1 = a1_all - a1_nn
    h1 = jnp.maximum(a1, 0.0)
    dt1 = jnp.where(a1 > 0.0, t1, 0.0)

    a2 = jax.lax.dot_general(h1, w2_ref[:], _CT) + b2_ref[:][None, :]
    t2 = jax.lax.dot_general(dt1, w2_ref[:], _CT)
    h2 = jnp.maximum(a2, 0.0)
    dt2 = jnp.where(a2 > 0.0, t2, 0.0)

    z0 = jax.lax.dot_general(h2, w3_ref[:], _CT) + b3_ref[:][None, :]
    gz = jax.lax.dot_general(dt2, w3_ref[:], _CT)
    zs = z0 + gz
    zs_ref[:] = zs

    h4 = jnp.maximum(jax.lax.dot_general(zs, w4_ref[:], _CT) + b4_ref[:][None, :], 0.0)
    h5 = jnp.maximum(jax.lax.dot_general(h4, w5_ref[:], _CT) + b5_ref[:][None, :], 0.0)
    xhat_ref[:] = jax.lax.dot_general(h5, w6_ref[:], _CT) + b6_ref[:][None, :]


def kernel(xs, W1, b1, W2, b2, W3, b3, W4, b4, W5, b5, W6, b6):
    d = xs.shape[1]
    call = pl.pallas_call(
        _body,
        out_shape=(
            jax.ShapeDtypeStruct((B, d), jnp.float32),
            jax.ShapeDtypeStruct((B, W3.shape[0]), jnp.float32),
        ),
    )
    x_hats, zs = call(xs, W1, b1, W2, b2, W3, b3, W4, b4, W5, b5, W6, b6)
    return (x_hats, zs)
